# baseline (speedup 1.0000x reference)
import functools

import jax
import jax.numpy as jnp
from jax import lax
from jax.experimental import pallas as pl
from jax.experimental.pallas import tpu as pltpu
from jax.experimental.pallas import tpu_sc as plsc

R = 128
N = 100000
TEMP_EPS = 1e-20

RB = 8
NRB = R // RB

L = 16
CHUNKS = N // L
NROWS2D = R * CHUNKS
NC, NS = 2, 16
CH_PER_W = NROWS2D // (NC * NS)
ZROWS = 5000
NDMA = CH_PER_W // ZROWS

OB = 512


def _argmax_body(l_ref, u_ref, idx_out):
    g = jnp.log(-jnp.log(u_ref[...] + TEMP_EPS) + TEMP_EPS)
    t = l_ref[...] + g
    cols = lax.broadcasted_iota(jnp.int32, t.shape, 1)
    t = jnp.where(cols < N, t, -jnp.inf)
    bmax = jnp.max(t, axis=1, keepdims=True)
    idx_out[...] = jnp.min(
        jnp.where(t == bmax, cols, jnp.int32(2**31 - 1)), axis=1, keepdims=True
    )


_argmax_call = pl.pallas_call(
    _argmax_body,
    out_shape=jax.ShapeDtypeStruct((R, 1), jnp.int32),
    grid=(NRB,),
    in_specs=[
        pl.BlockSpec((RB, N), lambda j: (j, 0)),
        pl.BlockSpec((RB, N), lambda j: (j, 0)),
    ],
    out_specs=pl.BlockSpec((RB, 1), lambda j: (j, 0)),
    compiler_params=pltpu.CompilerParams(
        dimension_semantics=("arbitrary",),
    ),
)


ROWS_PER_CORE = R // NC


def _sc_onehot_body(idx_hbm, eye_hbm, zin_hbm, out_hbm,
                    zbuf, idx_v, chunk_v, off_v, src,
                    zsem, fsem, gsem, osem):
    c = lax.axis_index("c")
    s = lax.axis_index("s")
    wid = c * NS + s

    pltpu.async_copy(zin_hbm, zbuf, zsem).wait()
    base = wid * CH_PER_W
    copies = [
        pltpu.async_copy(zbuf, out_hbm.at[pl.ds(base + k * ZROWS, ZROWS)], fsem)
        for k in range(NDMA)
    ]
    for cp in copies:
        cp.wait()

    plsc.subcore_barrier()

    @pl.when(s == 0)
    def _():
        pltpu.sync_copy(
            idx_hbm.at[pl.ds(c * ROWS_PER_CORE, ROWS_PER_CORE)], idx_v
        )
        lane = lax.iota(jnp.int32, L)
        for i in range(ROWS_PER_CORE // L):
            idxv = idx_v[pl.ds(i * L, L)]
            rows = c * ROWS_PER_CORE + i * L + lane
            chunk_v[pl.ds(i * L, L)] = rows * CHUNKS + (idxv >> 4)
            off_v[pl.ds(i * L, L)] = idxv & (L - 1)
        pltpu.async_copy(eye_hbm.at[off_v], src, gsem).wait()
        pltpu.async_copy(src, out_hbm.at[chunk_v], osem).wait()


@functools.lru_cache(maxsize=1)
def _sc_onehot_call():
    return pl.kernel(
        _sc_onehot_body,
        out_type=jax.ShapeDtypeStruct((NROWS2D, L), jnp.float32),
        mesh=plsc.VectorSubcoreMesh(
            core_axis_name="c", subcore_axis_name="s", num_cores=NC
        ),
        scratch_types=[
            pltpu.VMEM((ZROWS, L), jnp.float32),
            pltpu.VMEM((ROWS_PER_CORE,), jnp.int32),
            pltpu.VMEM((ROWS_PER_CORE,), jnp.int32),
            pltpu.VMEM((ROWS_PER_CORE,), jnp.int32),
            pltpu.VMEM((ROWS_PER_CORE, L), jnp.float32),
            pltpu.SemaphoreType.DMA,
            pltpu.SemaphoreType.DMA,
            pltpu.SemaphoreType.DMA,
            pltpu.SemaphoreType.DMA,
        ],
        compiler_params=pltpu.CompilerParams(use_tc_tiling_on_sc=False),
    )


def kernel(logits, uniform_noise):
    idx = _argmax_call(logits, uniform_noise)
    eye = jnp.eye(L, dtype=jnp.float32)
    zin = jnp.zeros((ZROWS, L), jnp.float32)
    out2d = _sc_onehot_call()(idx.reshape(R), eye, zin)
    return out2d.reshape(R, N)


# speedup vs baseline: 1.5064x; 1.5064x over previous
import functools

import jax
import jax.numpy as jnp
from jax import lax
from jax.experimental import pallas as pl
from jax.experimental.pallas import tpu as pltpu
from jax.experimental.pallas import tpu_sc as plsc

R = 128
N = 100000
TEMP_EPS = 1e-20

RB = 8
NRB = R // RB

L = 16
CHUNKS = N // L
NROWS2D = R * CHUNKS
NC, NS = 2, 16
CH_PER_W = NROWS2D // (NC * NS)
ZROWS = 5000
NDMA = CH_PER_W // ZROWS

OB = 512


def _onehot_fused_body(l_ref, u_ref, out_ref):
    g = jnp.log(-jnp.log(u_ref[...] + TEMP_EPS) + TEMP_EPS)
    t = l_ref[...] + g
    cols = lax.broadcasted_iota(jnp.int32, t.shape, 1)
    t = jnp.where(cols < N, t, -jnp.inf)
    bmax = jnp.max(t, axis=1, keepdims=True)
    bidx = jnp.min(
        jnp.where(t == bmax, cols, jnp.int32(2**31 - 1)), axis=1, keepdims=True
    )
    out_ref[...] = (cols == bidx).astype(jnp.float32)


_onehot_fused_call = pl.pallas_call(
    _onehot_fused_body,
    out_shape=jax.ShapeDtypeStruct((R, N), jnp.float32),
    grid=(NRB,),
    in_specs=[
        pl.BlockSpec((RB, N), lambda j: (j, 0)),
        pl.BlockSpec((RB, N), lambda j: (j, 0)),
    ],
    out_specs=pl.BlockSpec((RB, N), lambda j: (j, 0)),
    compiler_params=pltpu.CompilerParams(
        dimension_semantics=("arbitrary",),
    ),
)


ROWS_PER_CORE = R // NC


def _sc_onehot_body(idx_hbm, eye_hbm, zin_hbm, out_hbm,
                    zbuf, idx_v, chunk_v, off_v, src,
                    zsem, fsem, gsem, osem):
    c = lax.axis_index("c")
    s = lax.axis_index("s")
    wid = c * NS + s

    pltpu.async_copy(zin_hbm, zbuf, zsem).wait()
    base = wid * CH_PER_W
    copies = [
        pltpu.async_copy(zbuf, out_hbm.at[pl.ds(base + k * ZROWS, ZROWS)], fsem)
        for k in range(NDMA)
    ]
    for cp in copies:
        cp.wait()

    plsc.subcore_barrier()

    @pl.when(s == 0)
    def _():
        pltpu.sync_copy(
            idx_hbm.at[pl.ds(c * ROWS_PER_CORE, ROWS_PER_CORE)], idx_v
        )
        lane = lax.iota(jnp.int32, L)
        for i in range(ROWS_PER_CORE // L):
            idxv = idx_v[pl.ds(i * L, L)]
            rows = c * ROWS_PER_CORE + i * L + lane
            chunk_v[pl.ds(i * L, L)] = rows * CHUNKS + (idxv >> 4)
            off_v[pl.ds(i * L, L)] = idxv & (L - 1)
        pltpu.async_copy(eye_hbm.at[off_v], src, gsem).wait()
        pltpu.async_copy(src, out_hbm.at[chunk_v], osem).wait()


@functools.lru_cache(maxsize=1)
def _sc_onehot_call():
    return pl.kernel(
        _sc_onehot_body,
        out_type=jax.ShapeDtypeStruct((NROWS2D, L), jnp.float32),
        mesh=plsc.VectorSubcoreMesh(
            core_axis_name="c", subcore_axis_name="s", num_cores=NC
        ),
        scratch_types=[
            pltpu.VMEM((ZROWS, L), jnp.float32),
            pltpu.VMEM((ROWS_PER_CORE,), jnp.int32),
            pltpu.VMEM((ROWS_PER_CORE,), jnp.int32),
            pltpu.VMEM((ROWS_PER_CORE,), jnp.int32),
            pltpu.VMEM((ROWS_PER_CORE, L), jnp.float32),
            pltpu.SemaphoreType.DMA,
            pltpu.SemaphoreType.DMA,
            pltpu.SemaphoreType.DMA,
            pltpu.SemaphoreType.DMA,
        ],
        compiler_params=pltpu.CompilerParams(use_tc_tiling_on_sc=False),
    )


def kernel(logits, uniform_noise):
    return _onehot_fused_call(logits, uniform_noise)


# speedup vs baseline: 1.5247x; 1.0121x over previous
import jax
import jax.numpy as jnp
from jax import lax
from jax.experimental import pallas as pl
from jax.experimental.pallas import tpu as pltpu

R = 128
N = 100000
TEMP_EPS = 1e-20

RB = 8
NRB = R // RB


def _onehot_fused_body(l_ref, u_ref, out_ref):
    g = jnp.log(-jnp.log(u_ref[...] + TEMP_EPS) + TEMP_EPS)
    t = l_ref[...] + g
    cols = lax.broadcasted_iota(jnp.int32, t.shape, 1)
    t = jnp.where(cols < N, t, -jnp.inf)
    bmax = jnp.max(t, axis=1, keepdims=True)
    bidx = jnp.min(
        jnp.where(t == bmax, cols, jnp.int32(2**31 - 1)), axis=1, keepdims=True
    )
    out_ref[...] = (cols == bidx).astype(jnp.float32)


_onehot_fused_call = pl.pallas_call(
    _onehot_fused_body,
    out_shape=jax.ShapeDtypeStruct((R, N), jnp.float32),
    grid=(NRB,),
    in_specs=[
        pl.BlockSpec((RB, N), lambda j: (j, 0)),
        pl.BlockSpec((RB, N), lambda j: (j, 0)),
    ],
    out_specs=pl.BlockSpec((RB, N), lambda j: (j, 0)),
    compiler_params=pltpu.CompilerParams(
        dimension_semantics=("arbitrary",),
    ),
)


def kernel(logits, uniform_noise):
    return _onehot_fused_call(logits, uniform_noise)
